# baseline (device time: 72228 ns/iter reference)
import jax
import jax.numpy as jnp
from jax import lax
from jax.experimental import pallas as pl
from jax.experimental.pallas import tpu as pltpu

N_DEV = 8

MASK_DUAL = {1: 3, 3: 2, 4: 4}
PART_ORDERS = ((1, 3, 4), (3, 4, 1), (4, 1, 3))

(R0S0A, R0S0B, R0S1A, R0S1B, R1S0, R1S1, R2,
 AG0, AG1A, AG1B, AG2A, AG2B, AG2C1, AG2C2) = range(14)
N_XCHG = 14


def _keep_bit(i, dual):
    b = jnp.int32(0)
    for bit in range(3):
        if (dual >> bit) & 1:
            b = b ^ ((i >> bit) & 1)
    return b


def kernel(x):
    m, n = x.shape
    units = m // 128
    per = [units // 3 + (1 if p < units % 3 else 0) for p in range(3)]
    part_sz = [128 * u for u in per]
    part_base = [0, part_sz[0], part_sz[0] + part_sz[1]]

    slot_base = []
    off = 0
    for p in range(3):
        slots = []
        for k in range(3):
            slots.append(off)
            off += part_sz[p] >> (k + 1)
        slot_base.append(tuple(slots))
    recv_rows = off

    def body(x_ref, out_ref, stage_ref, rsr_ref, send_sems, recv_sems):
        my = lax.axis_index("i").astype(jnp.int32)

        barrier_sem = pltpu.get_barrier_semaphore()
        for mask in MASK_DUAL:
            pl.semaphore_signal(
                barrier_sem, inc=1,
                device_id=(my ^ mask,), device_id_type=pl.DeviceIdType.MESH,
            )
        pl.semaphore_wait(barrier_sem, 3)

        P = []
        for p in range(3):
            M = PART_ORDERS[p]
            b = [_keep_bit(my, MASK_DUAL[M[k]]) for k in range(3)]
            sz = part_sz[p]
            H, Q, E = sz >> 1, sz >> 2, sz >> 3
            K0 = part_base[p] + b[0] * H
            S0 = part_base[p] + (1 - b[0]) * H
            K1 = K0 + b[1] * Q
            S1 = K0 + (1 - b[1]) * Q
            K2 = K1 + b[2] * E
            S2 = K1 + (1 - b[2]) * E
            R0, R1, R2s = slot_base[p]
            P.append(dict(
                M=M, b=b, H=H, Q=Q, E=E,
                K0=K0, S0=S0, K1=K1, S1=S1, K2=K2, S2=S2,
                r0_fwd=R0 + (1 - b[1]) * Q,
                r0_s2=R0 + b[1] * Q + (1 - b[2]) * E,
                r0_k2=R0 + b[1] * Q + b[2] * E,
                r1_s2=R1 + (1 - b[2]) * E,
                r1_k2=R1 + b[2] * E,
                R2=R2s,
            ))

        d = [[None] * N_XCHG for _ in range(3)]

        def xchg(p, idx, src_ref, src_off, dst_ref, dst_off, rows, mask):
            r = pltpu.make_async_remote_copy(
                src_ref=src_ref.at[pl.ds(src_off, rows), :],
                dst_ref=dst_ref.at[pl.ds(dst_off, rows), :],
                send_sem=send_sems.at[p, idx],
                recv_sem=recv_sems.at[p, idx],
                device_id=(my ^ mask,),
                device_id_type=pl.DeviceIdType.MESH,
            )
            r.start()
            d[p][idx] = r

        def cast(dst_off, rows):
            stage_ref[pl.ds(dst_off, rows), :] = x_ref[
                pl.ds(dst_off, rows), :
            ].astype(jnp.bfloat16)

        def add(dst_off, slot_off, rows):
            x_ref[pl.ds(dst_off, rows), :] = (
                x_ref[pl.ds(dst_off, rows), :]
                + rsr_ref[pl.ds(slot_off, rows), :].astype(jnp.float32)
            )

        def settle(abs_off, rows):
            out_ref[pl.ds(abs_off, rows), :] = stage_ref[
                pl.ds(abs_off, rows), :
            ].astype(jnp.float32)

        for p, s in enumerate(P):
            b1, b2, Q, E = s["b"][1], s["b"][2], s["Q"], s["E"]
            fwd = s["S0"] + (1 - b1) * Q
            cast(fwd, Q)
            xchg(p, R0S0A, stage_ref, fwd + (1 - b2) * E,
                 rsr_ref, s["r0_fwd"] + (1 - b2) * E, E, s["M"][0])
            xchg(p, R0S0B, stage_ref, fwd + b2 * E,
                 rsr_ref, s["r0_fwd"] + b2 * E, E, s["M"][0])
        for p, s in enumerate(P):
            b1, b2, Q, E = s["b"][1], s["b"][2], s["Q"], s["E"]
            late = s["S0"] + b1 * Q
            cast(late, Q)
            xchg(p, R0S1A, stage_ref, late + (1 - b2) * E,
                 rsr_ref, s["r0_s2"], E, s["M"][0])
            xchg(p, R0S1B, stage_ref, late + b2 * E,
                 rsr_ref, s["r0_k2"], E, s["M"][0])
        for p, s in enumerate(P):
            b2, E = s["b"][2], s["E"]
            d[p][R0S0A].wait_recv()
            add(s["S1"] + (1 - b2) * E, s["r0_fwd"] + (1 - b2) * E, E)
            cast(s["S1"] + (1 - b2) * E, E)
            xchg(p, R1S0, stage_ref, s["S1"] + (1 - b2) * E,
                 rsr_ref, s["r1_s2"], E, s["M"][1])
        for p, s in enumerate(P):
            b2, E = s["b"][2], s["E"]
            d[p][R0S0B].wait_recv()
            add(s["S1"] + b2 * E, s["r0_fwd"] + b2 * E, E)
            cast(s["S1"] + b2 * E, E)
            xchg(p, R1S1, stage_ref, s["S1"] + b2 * E,
                 rsr_ref, s["r1_k2"], E, s["M"][1])
        for p, s in enumerate(P):
            E = s["E"]
            d[p][R0S1A].wait_recv()
            add(s["S2"], s["r0_s2"], E)
            d[p][R1S0].wait_recv()
            add(s["S2"], s["r1_s2"], E)
            cast(s["S2"], E)
            xchg(p, R2, stage_ref, s["S2"], rsr_ref, s["R2"], E, s["M"][2])
        for p, s in enumerate(P):
            E = s["E"]
            d[p][R0S1B].wait_recv()
            add(s["K2"], s["r0_k2"], E)
            d[p][R1S1].wait_recv()
            add(s["K2"], s["r1_k2"], E)

        for p, s in enumerate(P):
            E, K2, M = s["E"], s["K2"], s["M"]
            d[p][R2].wait_recv()
            add(K2, s["R2"], E)
            cast(K2, E)
            xchg(p, AG0, stage_ref, K2, stage_ref, K2, E, M[2])
            xchg(p, AG1A, stage_ref, K2, stage_ref, K2, E, M[1])
            xchg(p, AG2A, stage_ref, K2, stage_ref, K2, E, M[0])
            out_ref[pl.ds(K2, E), :] = x_ref[pl.ds(K2, E), :]
        for p, s in enumerate(P):
            d[p][AG0].wait_recv()
            xchg(p, AG1B, stage_ref, s["S2"], stage_ref, s["S2"], s["E"], s["M"][1])
            xchg(p, AG2B, stage_ref, s["S2"], stage_ref, s["S2"], s["E"], s["M"][0])
            settle(s["S2"], s["E"])
        for p, s in enumerate(P):
            b2, E = s["b"][2], s["E"]
            d[p][AG1A].wait_recv()
            xchg(p, AG2C1, stage_ref, s["S1"] + b2 * E,
                 stage_ref, s["S1"] + b2 * E, E, s["M"][0])
            d[p][AG1B].wait_recv()
            xchg(p, AG2C2, stage_ref, s["S1"] + (1 - b2) * E,
                 stage_ref, s["S1"] + (1 - b2) * E, E, s["M"][0])
            settle(s["S1"], s["Q"])
        for p, s in enumerate(P):
            b1, b2, Q, E = s["b"][1], s["b"][2], s["Q"], s["E"]
            d[p][AG2A].wait_recv()
            settle(s["S0"] + b1 * Q + b2 * E, E)
            d[p][AG2B].wait_recv()
            settle(s["S0"] + b1 * Q + (1 - b2) * E, E)
        for p, s in enumerate(P):
            b1, b2, Q, E = s["b"][1], s["b"][2], s["Q"], s["E"]
            d[p][AG2C1].wait_recv()
            settle(s["S0"] + (1 - b1) * Q + b2 * E, E)
            d[p][AG2C2].wait_recv()
            settle(s["S0"] + (1 - b1) * Q + (1 - b2) * E, E)
        for p in range(3):
            for idx in range(N_XCHG):
                d[p][idx].wait_send()

    return pl.pallas_call(
        body,
        out_shape=jax.ShapeDtypeStruct((m, n), x.dtype),
        in_specs=[pl.BlockSpec(memory_space=pltpu.VMEM)],
        out_specs=pl.BlockSpec(memory_space=pltpu.VMEM),
        scratch_shapes=[
            pltpu.VMEM((m, n), jnp.bfloat16),
            pltpu.VMEM((recv_rows, n), jnp.bfloat16),
            pltpu.SemaphoreType.DMA((3, N_XCHG)),
            pltpu.SemaphoreType.DMA((3, N_XCHG)),
        ],
        compiler_params=pltpu.CompilerParams(collective_id=0),
    )(x)
